# baseline (speedup 1.0000x reference)
import jax
import jax.numpy as jnp
from jax import lax
from jax.experimental import pallas as pl
from jax.experimental.pallas import tpu as pltpu
from jax.experimental.pallas import tpu_sc as plsc

B = 1024
S = 1001
E = 64
NC = 2
NS = 16
NW = NC * NS
ROWS_PER_W = B // NW
NCHUNK = 8
CW = 126
CG = 128
S_PAD = 1016
TAIL = S - (NCHUNK - 1) * CW
NBUF = 4
DEPTH = 3


def _emb_body(comb_hbm, pos_hbm, tab, out, comb0, comb1, pos_v,
              b0, b1, b2, b3, gs0, gs1, gs2, gs3, ss0, ss1, ss2, ss3,
              cs0, cs1):
    wid = lax.axis_index("s") * NC + lax.axis_index("c")
    base = wid * ROWS_PER_W
    bufs = [b0, b1, b2, b3]
    gsems = [gs0, gs1, gs2, gs3]
    ssems = [ss0, ss1, ss2, ss3]

    pltpu.sync_copy(pos_hbm, pos_v)

    ri = lax.iota(jnp.int32, 16)
    col63 = jnp.full((16,), E - 1, jnp.int32)

    def drain_store(p, rows, b):
        pltpu.make_async_copy(out.at[b].at[pl.ds(0, rows)],
                              bufs[p].at[pl.ds(0, rows)], ssems[p]).wait()

    def do_row(b, comb, guard):
        idx = comb.at[0]
        dstb = comb.at[1]

        def guarded_drain(p, rows, b):
            if guard is None:
                drain_store(p, rows, b)
            else:
                @pl.when(guard)
                def _():
                    drain_store(p, rows, b)

        gathers = {}
        for j in range(DEPTH):
            guarded_drain(j, CW, b)
            gathers[j] = pltpu.async_copy(tab.at[idx.at[j]], bufs[j], gsems[j])

        for j in range(NCHUNK):
            gathers[j].wait()
            bufp = bufs[j % NBUF]

            def add_pos(i, c, _j=j, _bufp=bufp):
                for cc in range(E // 16):
                    sl = pl.ds(cc * 16, 16)
                    _bufp[i, sl] = _bufp[i, sl] + pos_v[_j * CW + i, sl]
                return c

            lax.fori_loop(0, CG, add_pos, 0, unroll=4)

            for t in range(CG // 16):
                dv = plsc.bitcast(dstb[j, pl.ds(t * 16, 16)], jnp.float32)
                plsc.addupdate_scatter(bufp, [t * 16 + ri, col63], dv)

            rows = CW if j + 1 < NCHUNK else TAIL
            pltpu.async_copy(bufp.at[pl.ds(0, rows)],
                             out.at[b].at[pl.ds(j * CW, rows)], ssems[j % NBUF])

            jj = j + DEPTH
            if jj < NCHUNK:
                p = jj % NBUF
                if jj == DEPTH:
                    guarded_drain(p, TAIL, b)
                else:
                    drain_store(p, CW, b)
                gathers[jj] = pltpu.async_copy(tab.at[idx.at[jj]],
                                               bufs[p], gsems[p])

    pltpu.sync_copy(comb_hbm.at[base], comb0)
    pltpu.async_copy(comb_hbm.at[base + 1], comb1, cs1)

    def pair_body(g, carry):
        even = base + 2 * g

        @pl.when(g > 0)
        def _():
            pltpu.make_async_copy(comb_hbm.at[even], comb0, cs0).wait()

        do_row(even, comb0, guard=g > 0)

        @pl.when(g < ROWS_PER_W // 2 - 1)
        def _():
            pltpu.async_copy(comb_hbm.at[even + 2], comb0, cs0)

        pltpu.make_async_copy(comb_hbm.at[even + 1], comb1, cs1).wait()
        do_row(even + 1, comb1, guard=None)

        @pl.when(g < ROWS_PER_W // 2 - 1)
        def _():
            pltpu.async_copy(comb_hbm.at[even + 3], comb1, cs1)

        return carry

    lax.fori_loop(0, ROWS_PER_W // 2, pair_body, 0)

    last = base + ROWS_PER_W - 1
    for p in range(NBUF - 1):
        drain_store(p, CW, last)
    drain_store(NBUF - 1, TAIL, last)


def kernel(used_symbols, distribution, pos_encoding, symbol_embeddings):
    u = used_symbols[:, :S].astype(jnp.int32)
    u_pad = jnp.pad(u, ((0, 0), (0, S_PAD - S)))
    idx3 = jnp.stack([u_pad[:, j * CW:j * CW + CG]
                      for j in range(NCHUNK)], axis=1)
    d_pad = jnp.pad(distribution, ((0, 0), (0, S_PAD - S)))
    dist3 = jnp.stack([d_pad[:, j * CW:j * CW + CG]
                       for j in range(NCHUNK)], axis=1)
    comb = jnp.stack(
        [idx3, lax.bitcast_convert_type(dist3, jnp.int32)], axis=1)
    pos_pad = jnp.pad(pos_encoding, ((0, S_PAD - S), (0, 0)))
    tab = jnp.pad(symbol_embeddings, ((0, 0), (0, 1)))

    mesh = plsc.VectorSubcoreMesh(core_axis_name="c", subcore_axis_name="s")
    run = pl.kernel(
        _emb_body,
        out_type=jax.ShapeDtypeStruct((B, S, E), jnp.float32),
        mesh=mesh,
        scratch_types=[
            pltpu.VMEM((2, NCHUNK, CG), jnp.int32),
            pltpu.VMEM((2, NCHUNK, CG), jnp.int32),
            pltpu.VMEM((S_PAD, E), jnp.float32),
            pltpu.VMEM((CG, E), jnp.float32),
            pltpu.VMEM((CG, E), jnp.float32),
            pltpu.VMEM((CG, E), jnp.float32),
            pltpu.VMEM((CG, E), jnp.float32),
            pltpu.SemaphoreType.DMA,
            pltpu.SemaphoreType.DMA,
            pltpu.SemaphoreType.DMA,
            pltpu.SemaphoreType.DMA,
            pltpu.SemaphoreType.DMA,
            pltpu.SemaphoreType.DMA,
            pltpu.SemaphoreType.DMA,
            pltpu.SemaphoreType.DMA,
            pltpu.SemaphoreType.DMA,
            pltpu.SemaphoreType.DMA,
        ],
        compiler_params=pltpu.CompilerParams(use_tc_tiling_on_sc=False,
                                             needs_layout_passes=False),
    )
    return run(comb, pos_pad, tab)


# speedup vs baseline: 1.0081x; 1.0081x over previous
import jax
import jax.numpy as jnp
from jax import lax
from jax.experimental import pallas as pl
from jax.experimental.pallas import tpu as pltpu
from jax.experimental.pallas import tpu_sc as plsc

B = 1024
S = 1001
E = 64
NC = 2
NS = 16
NW = NC * NS
ROWS_PER_W = B // NW
NCHUNK = 8
CW = 126
CG = 128
S_PAD = 1016
TAIL = S - (NCHUNK - 1) * CW
NBUF = 4
DEPTH = 3


def _emb_body(comb_hbm, pos_hbm, tab, out, comb0, comb1, pos_v,
              b0, b1, b2, b3, gs0, gs1, gs2, gs3, ss0, ss1, ss2, ss3,
              cs0, cs1):
    wid = lax.axis_index("s") * NC + lax.axis_index("c")
    base = wid * ROWS_PER_W
    bufs = [b0, b1, b2, b3]
    gsems = [gs0, gs1, gs2, gs3]
    ssems = [ss0, ss1, ss2, ss3]

    pltpu.sync_copy(pos_hbm, pos_v)

    ri = lax.iota(jnp.int32, 16)
    col63 = jnp.full((16,), E - 1, jnp.int32)

    def drain_store(p, rows, b):
        pltpu.make_async_copy(out.at[b].at[pl.ds(0, rows)],
                              bufs[p].at[pl.ds(0, rows)], ssems[p]).wait()

    def do_row(b, comb, guard):
        idx = comb.at[0]
        dstb = comb.at[1]

        def guarded_drain(p, rows, b):
            if guard is None:
                drain_store(p, rows, b)
            else:
                @pl.when(guard)
                def _():
                    drain_store(p, rows, b)

        gathers = {}
        for j in range(DEPTH):
            guarded_drain(j, CW, b)
            gathers[j] = pltpu.async_copy(tab.at[idx.at[j]], bufs[j], gsems[j])

        for j in range(NCHUNK):
            gathers[j].wait()
            bufp = bufs[j % NBUF]

            def add_pos(i, c, _j=j, _bufp=bufp):
                for cc in range(E // 16):
                    sl = pl.ds(cc * 16, 16)
                    _bufp[i, sl] = _bufp[i, sl] + pos_v[_j * CW + i, sl]
                return c

            lax.fori_loop(0, CG, add_pos, 0)

            for t in range(CG // 16):
                dv = plsc.bitcast(dstb[j, pl.ds(t * 16, 16)], jnp.float32)
                plsc.addupdate_scatter(bufp, [t * 16 + ri, col63], dv)

            rows = CW if j + 1 < NCHUNK else TAIL
            pltpu.async_copy(bufp.at[pl.ds(0, rows)],
                             out.at[b].at[pl.ds(j * CW, rows)], ssems[j % NBUF])

            jj = j + DEPTH
            if jj < NCHUNK:
                p = jj % NBUF
                if jj == DEPTH:
                    guarded_drain(p, TAIL, b)
                else:
                    drain_store(p, CW, b)
                gathers[jj] = pltpu.async_copy(tab.at[idx.at[jj]],
                                               bufs[p], gsems[p])

    pltpu.sync_copy(comb_hbm.at[base], comb0)
    pltpu.async_copy(comb_hbm.at[base + 1], comb1, cs1)

    def pair_body(g, carry):
        even = base + 2 * g

        @pl.when(g > 0)
        def _():
            pltpu.make_async_copy(comb_hbm.at[even], comb0, cs0).wait()

        do_row(even, comb0, guard=g > 0)

        @pl.when(g < ROWS_PER_W // 2 - 1)
        def _():
            pltpu.async_copy(comb_hbm.at[even + 2], comb0, cs0)

        pltpu.make_async_copy(comb_hbm.at[even + 1], comb1, cs1).wait()
        do_row(even + 1, comb1, guard=None)

        @pl.when(g < ROWS_PER_W // 2 - 1)
        def _():
            pltpu.async_copy(comb_hbm.at[even + 3], comb1, cs1)

        return carry

    lax.fori_loop(0, ROWS_PER_W // 2, pair_body, 0)

    last = base + ROWS_PER_W - 1
    for p in range(NBUF - 1):
        drain_store(p, CW, last)
    drain_store(NBUF - 1, TAIL, last)


def kernel(used_symbols, distribution, pos_encoding, symbol_embeddings):
    u = used_symbols[:, :S].astype(jnp.int32)
    u_pad = jnp.pad(u, ((0, 0), (0, S_PAD - S)))
    idx3 = jnp.stack([u_pad[:, j * CW:j * CW + CG]
                      for j in range(NCHUNK)], axis=1)
    d_pad = jnp.pad(distribution, ((0, 0), (0, S_PAD - S)))
    dist3 = jnp.stack([d_pad[:, j * CW:j * CW + CG]
                       for j in range(NCHUNK)], axis=1)
    comb = jnp.stack(
        [idx3, lax.bitcast_convert_type(dist3, jnp.int32)], axis=1)
    pos_pad = jnp.pad(pos_encoding, ((0, S_PAD - S), (0, 0)))
    tab = jnp.pad(symbol_embeddings, ((0, 0), (0, 1)))

    mesh = plsc.VectorSubcoreMesh(core_axis_name="c", subcore_axis_name="s")
    run = pl.kernel(
        _emb_body,
        out_type=jax.ShapeDtypeStruct((B, S, E), jnp.float32),
        mesh=mesh,
        scratch_types=[
            pltpu.VMEM((2, NCHUNK, CG), jnp.int32),
            pltpu.VMEM((2, NCHUNK, CG), jnp.int32),
            pltpu.VMEM((S_PAD, E), jnp.float32),
            pltpu.VMEM((CG, E), jnp.float32),
            pltpu.VMEM((CG, E), jnp.float32),
            pltpu.VMEM((CG, E), jnp.float32),
            pltpu.VMEM((CG, E), jnp.float32),
            pltpu.SemaphoreType.DMA,
            pltpu.SemaphoreType.DMA,
            pltpu.SemaphoreType.DMA,
            pltpu.SemaphoreType.DMA,
            pltpu.SemaphoreType.DMA,
            pltpu.SemaphoreType.DMA,
            pltpu.SemaphoreType.DMA,
            pltpu.SemaphoreType.DMA,
            pltpu.SemaphoreType.DMA,
            pltpu.SemaphoreType.DMA,
        ],
        compiler_params=pltpu.CompilerParams(use_tc_tiling_on_sc=False,
                                             needs_layout_passes=False),
    )
    return run(comb, pos_pad, tab)


# speedup vs baseline: 1.3136x; 1.3031x over previous
import jax
import jax.numpy as jnp
from jax import lax
from jax.experimental import pallas as pl
from jax.experimental.pallas import tpu as pltpu
from jax.experimental.pallas import tpu_sc as plsc

B = 1024
S = 1001
E = 64
NC = 2
NS = 16
NW = NC * NS
ROWS_PER_W = B // NW
GLEN = (336, 336, 329)
GOFF = (0, 336, 672)
NG = len(GLEN)


def _emb_body(used_hbm, dist_hbm, pos_hbm, tab, out,
              idx0, idx1, dsv0, dsv1, pos_v, b0, b1,
              gs0, gs1, ss0, ss1, cs0, cs1):
    wid = lax.axis_index("s") * NC + lax.axis_index("c")
    base = wid * ROWS_PER_W
    bufs = [b0, b1]
    gsems = [gs0, gs1]
    ssems = [ss0, ss1]

    pltpu.sync_copy(pos_hbm, pos_v)

    ri = lax.iota(jnp.int32, 16)
    col63 = jnp.full((16,), E - 1, jnp.int32)

    def drain_store(p, rows, b):
        pltpu.make_async_copy(out.at[b].at[pl.ds(0, rows)],
                              bufs[p].at[pl.ds(0, rows)], ssems[p]).wait()

    def stage_row(b, idx_v, dsv, sem):
        pltpu.async_copy(used_hbm.at[b].at[pl.ds(0, S)],
                         idx_v.at[pl.ds(0, S)], sem)
        pltpu.async_copy(dist_hbm.at[b], dsv.at[pl.ds(0, S)], sem)

    def wait_stage(b, idx_v, dsv, sem):
        pltpu.make_async_copy(used_hbm.at[b].at[pl.ds(0, S)],
                              idx_v.at[pl.ds(0, S)], sem).wait()
        pltpu.make_async_copy(dist_hbm.at[b], dsv.at[pl.ds(0, S)], sem).wait()

    def do_row(b, idx_v, dsv, bmap, guard):
        gathers = {}
        for g in range(2):
            p = bmap[g]
            rows_prev = GLEN[1] if g == 0 else GLEN[2]
            if guard is None:
                drain_store(p, rows_prev, b)
            else:
                @pl.when(guard)
                def _():
                    drain_store(p, rows_prev, b)
            gathers[g] = pltpu.async_copy(
                tab.at[idx_v.at[pl.ds(GOFF[g], GLEN[g])]],
                bufs[p].at[pl.ds(0, GLEN[g])], gsems[p])

        for g in range(NG):
            glen, goff = GLEN[g], GOFF[g]
            p = bmap[g]
            gathers[g].wait()
            bufp = bufs[p]

            def add_pos(i, c, _goff=goff, _bufp=bufp):
                for cc in range(E // 16):
                    sl = pl.ds(cc * 16, 16)
                    _bufp[i, sl] = _bufp[i, sl] + pos_v[_goff + i, sl]
                return c

            lax.fori_loop(0, glen, add_pos, 0)

            def add_dist(t, c, _goff=goff, _glen=glen, _bufp=bufp):
                rows = t * 16 + ri
                dv = dsv[pl.ds(_goff + t * 16, 16)]
                if _glen % 16 == 0:
                    plsc.addupdate_scatter(_bufp, [rows, col63], dv)
                else:
                    plsc.addupdate_scatter(_bufp, [rows, col63], dv,
                                           mask=rows < _glen)
                return c

            lax.fori_loop(0, (glen + 15) // 16, add_dist, 0)

            pltpu.async_copy(bufp.at[pl.ds(0, glen)],
                             out.at[b].at[pl.ds(goff, glen)], ssems[p])

            if g == 0:
                drain_store(bmap[0], GLEN[0], b)
                gathers[2] = pltpu.async_copy(
                    tab.at[idx_v.at[pl.ds(GOFF[2], GLEN[2])]],
                    bufs[bmap[2]].at[pl.ds(0, GLEN[2])], gsems[bmap[2]])

    stage_row(base, idx0, dsv0, cs0)
    stage_row(base + 1, idx1, dsv1, cs1)

    def pair_body(g, carry):
        even = base + 2 * g

        wait_stage(even, idx0, dsv0, cs0)
        do_row(even, idx0, dsv0, (0, 1, 0), guard=g > 0)

        @pl.when(g < ROWS_PER_W // 2 - 1)
        def _():
            stage_row(even + 2, idx0, dsv0, cs0)

        wait_stage(even + 1, idx1, dsv1, cs1)
        do_row(even + 1, idx1, dsv1, (1, 0, 1), guard=None)

        @pl.when(g < ROWS_PER_W // 2 - 1)
        def _():
            stage_row(even + 3, idx1, dsv1, cs1)

        return carry

    lax.fori_loop(0, ROWS_PER_W // 2, pair_body, 0)

    last = base + ROWS_PER_W - 1
    drain_store(0, GLEN[1], last)
    drain_store(1, GLEN[2], last)


def kernel(used_symbols, distribution, pos_encoding, symbol_embeddings):
    u = used_symbols.astype(jnp.int32)
    tab = jnp.pad(symbol_embeddings, ((0, 0), (0, 1)))

    mesh = plsc.VectorSubcoreMesh(core_axis_name="c", subcore_axis_name="s")
    run = pl.kernel(
        _emb_body,
        out_type=jax.ShapeDtypeStruct((B, S, E), jnp.float32),
        mesh=mesh,
        scratch_types=[
            pltpu.VMEM((1008,), jnp.int32),
            pltpu.VMEM((1008,), jnp.int32),
            pltpu.VMEM((1008,), jnp.float32),
            pltpu.VMEM((1008,), jnp.float32),
            pltpu.VMEM((S, E), jnp.float32),
            pltpu.VMEM((GLEN[0], E), jnp.float32),
            pltpu.VMEM((GLEN[0], E), jnp.float32),
            pltpu.SemaphoreType.DMA,
            pltpu.SemaphoreType.DMA,
            pltpu.SemaphoreType.DMA,
            pltpu.SemaphoreType.DMA,
            pltpu.SemaphoreType.DMA,
            pltpu.SemaphoreType.DMA,
        ],
        compiler_params=pltpu.CompilerParams(use_tc_tiling_on_sc=False,
                                             needs_layout_passes=False),
    )
    return run(u, distribution, pos_encoding, tab)


# speedup vs baseline: 1.8750x; 1.4274x over previous
import jax
import jax.numpy as jnp
from jax import lax
from jax.experimental import pallas as pl
from jax.experimental.pallas import tpu as pltpu
from jax.experimental.pallas import tpu_sc as plsc

B = 1024
S = 1001
E = 64
NC = 2
NS = 16
NW = NC * NS
S_PER_W = 32
NBUF = 4


def _emb_body(ut_hbm, dt_hbm, post_hbm, tabt_hbm, out,
              tabt_v, pos_sv, pos_col, idx_v, dsv, b0, b1, b2, b3,
              ss0, ss1, ss2, ss3, stg):
    wid = lax.axis_index("s") * NC + lax.axis_index("c")
    s0 = wid * S_PER_W
    scount = jnp.maximum(0, jnp.minimum(S_PER_W, S - s0))
    bufs = [b0, b1, b2, b3]
    ssems = [ss0, ss1, ss2, ss3]

    pltpu.sync_copy(tabt_hbm, tabt_v)
    pltpu.sync_copy(post_hbm.at[:, pl.ds(s0, S_PER_W)], pos_sv)

    def drain_store(p, s):
        pltpu.make_async_copy(out.at[s].at[p % (E // 8)],
                              bufs[p], ssems[p]).wait()

    ri = lax.iota(jnp.int32, 16)

    def s_body(s_loc, carry):
        s = s0 + s_loc
        c1 = pltpu.async_copy(ut_hbm.at[s], idx_v, stg)
        c2 = pltpu.async_copy(dt_hbm.at[s], dsv, stg)
        scol = ri * 0 + s_loc
        for t in range(E // 16):
            pos_col[pl.ds(t * 16, 16)] = plsc.load_gather(
                pos_sv, [t * 16 + ri, scol])
        c1.wait()
        c2.wait()

        for et in range(E // 8):
            p = et % NBUF
            if et < NBUF:
                @pl.when(s_loc > 0)
                def _():
                    drain_store(p, s)
            else:
                drain_store(p, s)
            bufp = bufs[p]

            def blk_body(blk, c, _et=et, _bufp=bufp):
                ivecs = [idx_v[pl.ds(blk * 128 + j * 16, 16)]
                         for j in range(8)]

                def e_body(e_i, c2):
                    e = jnp.int32(_et * 8) + e_i
                    esp = ri * 0 + e
                    psp = plsc.load_gather(pos_col, [esp])
                    for j in range(8):
                        g = plsc.load_gather(tabt_v, [esp, ivecs[j]])
                        _bufp[blk, e_i, pl.ds(j * 16, 16)] = g + psp
                    return c2

                lax.fori_loop(0, 8, e_body, 0)

                if _et == E // 8 - 1:
                    for j in range(8):
                        sl = pl.ds(j * 16, 16)
                        dv = dsv[pl.ds(blk * 128 + j * 16, 16)]
                        _bufp[blk, 7, sl] = _bufp[blk, 7, sl] + dv
                return c

            lax.fori_loop(0, 8, blk_body, 0)

            pltpu.async_copy(bufp, out.at[s].at[et], ssems[p])
        return carry

    lax.fori_loop(0, scount, s_body, 0)

    last = s0 + scount - 1

    @pl.when(scount > 0)
    def _():
        for p in range(NBUF):
            drain_store(p, last)


def kernel(used_symbols, distribution, pos_encoding, symbol_embeddings):
    ut = used_symbols.astype(jnp.int32).T
    dt = distribution.T
    post = jnp.pad(pos_encoding.T, ((0, 0), (0, NW * S_PER_W - S)))
    tabt = jnp.pad(symbol_embeddings, ((0, 0), (0, 1))).T

    mesh = plsc.VectorSubcoreMesh(core_axis_name="c", subcore_axis_name="s")
    run = pl.kernel(
        _emb_body,
        out_type=jax.ShapeDtypeStruct((S, E // 8, B // 128, 8, 128),
                                      jnp.float32),
        mesh=mesh,
        scratch_types=[
            pltpu.VMEM((E, S), jnp.float32),
            pltpu.VMEM((E, S_PER_W), jnp.float32),
            pltpu.VMEM((E,), jnp.float32),
            pltpu.VMEM((B,), jnp.int32),
            pltpu.VMEM((B,), jnp.float32),
            pltpu.VMEM((B // 128, 8, 128), jnp.float32),
            pltpu.VMEM((B // 128, 8, 128), jnp.float32),
            pltpu.VMEM((B // 128, 8, 128), jnp.float32),
            pltpu.VMEM((B // 128, 8, 128), jnp.float32),
            pltpu.SemaphoreType.DMA,
            pltpu.SemaphoreType.DMA,
            pltpu.SemaphoreType.DMA,
            pltpu.SemaphoreType.DMA,
            pltpu.SemaphoreType.DMA,
        ],
        compiler_params=pltpu.CompilerParams(use_tc_tiling_on_sc=False,
                                             needs_layout_passes=False),
    )
    out5 = run(ut, dt, post, tabt)
    return out5.transpose(2, 4, 0, 1, 3).reshape(B, S, E)
